# baseline (speedup 1.0000x reference)
import functools

import jax
import jax.numpy as jnp
from jax import lax
from jax.experimental import pallas as pl
from jax.experimental.pallas import tpu as pltpu
from jax.experimental.pallas import tpu_sc as plsc

_B, _C, _N, _K = 1, 128, 10000, 32
_COUT = 128

_NC = 2
_NS = 16
_NW = _NC * _NS
_NPAD = 10240
_CN = 4
_ROWS = _CN * _K
_LC = _C // 16

_QT0 = 10
_QT1 = 40 - _QT0
_NB0 = 16 * _QT0
_NB1 = 16 * _QT1


def _tc_embed_body(xt_ref, wt_ref, u_ref, v_ref):
    xt = xt_ref[...]
    w2t = wt_ref[_C:, :]
    at = wt_ref[:_C, :] - w2t
    u_ref[...] = jnp.dot(xt, at, preferred_element_type=jnp.float32)
    v_ref[...] = jnp.dot(xt, w2t, preferred_element_type=jnp.float32)


def _tc_embed(xt, wt):
    return pl.pallas_call(
        _tc_embed_body,
        out_shape=[
            jax.ShapeDtypeStruct((_N, _COUT), jnp.float32),
            jax.ShapeDtypeStruct((_N, _COUT), jnp.float32),
        ],
    )(xt, wt)


def _sc_body(u_hbm, v_hbm, ii_hbm, jj_hbm, b_hbm, out_hbm,
             iv0, iv1, iv2, iv3, jv0, jv1, jv2, jv3,
             ur0, ur1, vr0, vr1, ov0, ov1, bv,
             sem_i0, sem_i1, sem_i2, sem_i3,
             sem_g0, sem_g1, sem_o0, sem_o1):
    cid = lax.axis_index("c")
    sid = lax.axis_index("s")
    n0 = jnp.where(cid == 0, sid * _NB0, _NS * _NB0 + sid * _NB1)
    qt = jnp.where(cid == 0, _QT0, _QT1)
    pltpu.sync_copy(b_hbm, bv)
    bvecs = [bv[pl.ds(c * 16, 16)] for c in range(_LC)]
    neg = jnp.full((16,), -jnp.inf, jnp.float32)

    ivs = (iv0, iv1, iv2, iv3)
    jvs = (jv0, jv1, jv2, jv3)
    sem_i = (sem_i0, sem_i1, sem_i2, sem_i3)
    urs = (ur0, ur1)
    vrs = (vr0, vr1)
    ovs = (ov0, ov1)
    sem_g = (sem_g0, sem_g1)
    sem_o = (sem_o0, sem_o1)

    def fetch_idx(k, chunk):
        base = (n0 + chunk * _CN) * _K
        pltpu.async_copy(ii_hbm.at[pl.ds(base, _ROWS)], ivs[k], sem_i[k])
        pltpu.async_copy(jj_hbm.at[pl.ds(base, _ROWS)], jvs[k], sem_i[k])

    def wait_idx(k):
        pltpu.make_async_copy(
            ii_hbm.at[pl.ds(0, _ROWS)], ivs[k], sem_i[k]).wait()
        pltpu.make_async_copy(
            jj_hbm.at[pl.ds(0, _ROWS)], jvs[k], sem_i[k]).wait()

    def issue_gather(r, k):
        pltpu.async_copy(u_hbm.at[ivs[k]], urs[r], sem_g[r])
        pltpu.async_copy(v_hbm.at[jvs[k]], vrs[r], sem_g[r])

    def wait_gather(r, k):
        pltpu.make_async_copy(u_hbm.at[ivs[k]], urs[r], sem_g[r]).wait()
        pltpu.make_async_copy(v_hbm.at[jvs[k]], vrs[r], sem_g[r]).wait()

    def issue_out(r, chunk):
        pltpu.async_copy(
            ovs[r], out_hbm.at[pl.ds(n0 + chunk * _CN, _CN)], sem_o[r])

    def wait_out(r):
        pltpu.make_async_copy(
            ovs[r], out_hbm.at[pl.ds(0, _CN)], sem_o[r]).wait()

    def compute(r):
        ur, vr, ov = urs[r], vrs[r], ovs[r]
        for n in range(_CN):
            def kbody(kk, accs, n=n):
                row = n * _K + kk
                return tuple(
                    jnp.maximum(accs[c],
                                ur[row, pl.ds(c * 16, 16)]
                                + vr[row, pl.ds(c * 16, 16)])
                    for c in range(_LC))
            accs = lax.fori_loop(0, _K, kbody, (neg,) * _LC)
            for c in range(_LC):
                ov[n, pl.ds(c * 16, 16)] = jnp.maximum(
                    accs[c] + bvecs[c], 0.0)

    for k in range(4):
        fetch_idx(k, k)
    wait_idx(0)
    issue_gather(0, 0)
    wait_idx(1)
    issue_gather(1, 1)

    def body(t, carry):
        for j in range(4):
            g = 4 * t + j
            r = j % 2
            wait_gather(r, j)

            @pl.when(t < qt - 1)
            def _():
                fetch_idx(j, g + 4)

            if j >= 2:
                wait_out(r)
            else:
                @pl.when(t > 0)
                def _():
                    wait_out(r)

            compute(r)
            issue_out(r, g)

            def refill():
                k2 = (j + 2) % 4
                wait_idx(k2)
                issue_gather(r, k2)

            if j < 2:
                refill()
            else:
                pl.when(t < qt - 1)(refill)
        return carry

    lax.fori_loop(0, qt, body, 0)
    wait_out(0)
    wait_out(1)


def _sc_gather(u, v, ii, jj, b):
    mesh = plsc.VectorSubcoreMesh(core_axis_name="c", subcore_axis_name="s")
    fn = functools.partial(
        pl.kernel,
        out_type=jax.ShapeDtypeStruct((_NPAD, _COUT), jnp.float32),
        mesh=mesh,
        scratch_types=(
            [pltpu.VMEM((_ROWS,), jnp.int32)] * 4
            + [pltpu.VMEM((_ROWS,), jnp.int32)] * 4
            + [pltpu.VMEM((_ROWS, _COUT), jnp.float32)] * 2
            + [pltpu.VMEM((_ROWS, _COUT), jnp.float32)] * 2
            + [pltpu.VMEM((_CN, _COUT), jnp.float32)] * 2
            + [pltpu.VMEM((_COUT,), jnp.float32)]
            + [pltpu.SemaphoreType.DMA] * 8
        ),
    )(_sc_body)
    return fn(u, v, ii, jj, b)


def kernel(x, edge_index, W, b):
    xt = jnp.transpose(x.reshape(_C, _N))
    wt = jnp.transpose(W)
    u, v = _tc_embed(xt, wt)

    ei = edge_index.astype(jnp.int32)
    ii = ei[1].reshape(_N * _K)
    jj = ei[0].reshape(_N * _K)
    pad = _NPAD * _K - _N * _K
    ii = jnp.pad(ii, (0, pad))
    jj = jnp.pad(jj, (0, pad))

    out = _sc_gather(u, v, ii, jj, b)
    out = jnp.transpose(out[:_N])
    return out.reshape(_B, _COUT, _N, 1)


# speedup vs baseline: 1.0721x; 1.0721x over previous
import functools

import jax
import jax.numpy as jnp
from jax import lax
from jax.experimental import pallas as pl
from jax.experimental.pallas import tpu as pltpu
from jax.experimental.pallas import tpu_sc as plsc

_B, _C, _N, _K = 1, 128, 10000, 32
_COUT = 128

_NC = 2
_NS = 16
_NW = _NC * _NS
_NPAD = 10240
_CN = 4
_ROWS = _CN * _K
_LC = _C // 16

_QT0 = 30
_QT1 = 40 - _QT0
_NB0 = 16 * _QT0
_NB1 = 16 * _QT1


def _tc_embed_body(xt_ref, wt_ref, u_ref, v_ref):
    xt = xt_ref[...]
    w2t = wt_ref[_C:, :]
    at = wt_ref[:_C, :] - w2t
    u_ref[...] = jnp.dot(xt, at, preferred_element_type=jnp.float32)
    v_ref[...] = jnp.dot(xt, w2t, preferred_element_type=jnp.float32)


def _tc_embed(xt, wt):
    return pl.pallas_call(
        _tc_embed_body,
        out_shape=[
            jax.ShapeDtypeStruct((_N, _COUT), jnp.float32),
            jax.ShapeDtypeStruct((_N, _COUT), jnp.float32),
        ],
    )(xt, wt)


def _sc_body(u_hbm, v_hbm, ii_hbm, jj_hbm, b_hbm, out_hbm,
             iv0, iv1, iv2, iv3, jv0, jv1, jv2, jv3,
             ur0, ur1, vr0, vr1, ov0, ov1, bv,
             sem_i0, sem_i1, sem_i2, sem_i3,
             sem_g0, sem_g1, sem_o0, sem_o1):
    cid = lax.axis_index("c")
    sid = lax.axis_index("s")
    n0 = jnp.where(cid == 0, sid * _NB0, _NS * _NB0 + sid * _NB1)
    qt = jnp.where(cid == 0, _QT0, _QT1)
    pltpu.sync_copy(b_hbm, bv)
    bvecs = [bv[pl.ds(c * 16, 16)] for c in range(_LC)]
    neg = jnp.full((16,), -jnp.inf, jnp.float32)

    ivs = (iv0, iv1, iv2, iv3)
    jvs = (jv0, jv1, jv2, jv3)
    sem_i = (sem_i0, sem_i1, sem_i2, sem_i3)
    urs = (ur0, ur1)
    vrs = (vr0, vr1)
    ovs = (ov0, ov1)
    sem_g = (sem_g0, sem_g1)
    sem_o = (sem_o0, sem_o1)

    def fetch_idx(k, chunk):
        base = (n0 + chunk * _CN) * _K
        pltpu.async_copy(ii_hbm.at[pl.ds(base, _ROWS)], ivs[k], sem_i[k])
        pltpu.async_copy(jj_hbm.at[pl.ds(base, _ROWS)], jvs[k], sem_i[k])

    def wait_idx(k):
        pltpu.make_async_copy(
            ii_hbm.at[pl.ds(0, _ROWS)], ivs[k], sem_i[k]).wait()
        pltpu.make_async_copy(
            jj_hbm.at[pl.ds(0, _ROWS)], jvs[k], sem_i[k]).wait()

    def issue_gather(r, k):
        pltpu.async_copy(u_hbm.at[ivs[k]], urs[r], sem_g[r])
        pltpu.async_copy(v_hbm.at[jvs[k]], vrs[r], sem_g[r])

    def wait_gather(r, k):
        pltpu.make_async_copy(u_hbm.at[ivs[k]], urs[r], sem_g[r]).wait()
        pltpu.make_async_copy(v_hbm.at[jvs[k]], vrs[r], sem_g[r]).wait()

    def issue_out(r, chunk):
        pltpu.async_copy(
            ovs[r], out_hbm.at[pl.ds(n0 + chunk * _CN, _CN)], sem_o[r])

    def wait_out(r):
        pltpu.make_async_copy(
            ovs[r], out_hbm.at[pl.ds(0, _CN)], sem_o[r]).wait()

    def compute(r):
        ur, vr, ov = urs[r], vrs[r], ovs[r]
        for n in range(_CN):
            def kbody(kk, accs, n=n):
                row = n * _K + kk
                return tuple(
                    jnp.maximum(accs[c],
                                ur[row, pl.ds(c * 16, 16)]
                                + vr[row, pl.ds(c * 16, 16)])
                    for c in range(_LC))
            accs = lax.fori_loop(0, _K, kbody, (neg,) * _LC)
            for c in range(_LC):
                ov[n, pl.ds(c * 16, 16)] = jnp.maximum(
                    accs[c] + bvecs[c], 0.0)

    for k in range(4):
        fetch_idx(k, k)
    wait_idx(0)
    issue_gather(0, 0)
    wait_idx(1)
    issue_gather(1, 1)

    def body(t, carry):
        for j in range(4):
            g = 4 * t + j
            r = j % 2
            wait_gather(r, j)

            @pl.when(t < qt - 1)
            def _():
                fetch_idx(j, g + 4)

            if j >= 2:
                wait_out(r)
            else:
                @pl.when(t > 0)
                def _():
                    wait_out(r)

            compute(r)
            issue_out(r, g)

            def refill():
                k2 = (j + 2) % 4
                wait_idx(k2)
                issue_gather(r, k2)

            if j < 2:
                refill()
            else:
                pl.when(t < qt - 1)(refill)
        return carry

    lax.fori_loop(0, qt, body, 0)
    wait_out(0)
    wait_out(1)


def _sc_gather(u, v, ii, jj, b):
    mesh = plsc.VectorSubcoreMesh(core_axis_name="c", subcore_axis_name="s")
    fn = functools.partial(
        pl.kernel,
        out_type=jax.ShapeDtypeStruct((_NPAD, _COUT), jnp.float32),
        mesh=mesh,
        scratch_types=(
            [pltpu.VMEM((_ROWS,), jnp.int32)] * 4
            + [pltpu.VMEM((_ROWS,), jnp.int32)] * 4
            + [pltpu.VMEM((_ROWS, _COUT), jnp.float32)] * 2
            + [pltpu.VMEM((_ROWS, _COUT), jnp.float32)] * 2
            + [pltpu.VMEM((_CN, _COUT), jnp.float32)] * 2
            + [pltpu.VMEM((_COUT,), jnp.float32)]
            + [pltpu.SemaphoreType.DMA] * 8
        ),
    )(_sc_body)
    return fn(u, v, ii, jj, b)


def kernel(x, edge_index, W, b):
    xt = jnp.transpose(x.reshape(_C, _N))
    wt = jnp.transpose(W)
    u, v = _tc_embed(xt, wt)

    ei = edge_index.astype(jnp.int32)
    ii = ei[1].reshape(_N * _K)
    jj = ei[0].reshape(_N * _K)
    pad = _NPAD * _K - _N * _K
    ii = jnp.pad(ii, (0, pad))
    jj = jnp.pad(jj, (0, pad))

    out = _sc_gather(u, v, ii, jj, b)
    out = jnp.transpose(out[:_N])
    return out.reshape(_B, _COUT, _N, 1)


# speedup vs baseline: 1.0887x; 1.0155x over previous
import functools

import jax
import jax.numpy as jnp
from jax import lax
from jax.experimental import pallas as pl
from jax.experimental.pallas import tpu as pltpu
from jax.experimental.pallas import tpu_sc as plsc

_B, _C, _N, _K = 1, 128, 10000, 32
_COUT = 128

_NC = 2
_NS = 16
_NW = _NC * _NS
_NPAD = 10240
_CN = 2
_ROWS = _CN * _K
_LC = _C // 16

_QT0 = 30
_QT1 = 40 - _QT0
_NB0 = 16 * _QT0
_NB1 = 16 * _QT1
_NBMAX = 16 * max(_QT0, _QT1)


def _tc_embed_body(xt_ref, wt_ref, u_ref, v_ref):
    xt = xt_ref[...]
    w2t = wt_ref[_C:, :]
    at = wt_ref[:_C, :] - w2t
    u_ref[...] = jnp.dot(xt, at, preferred_element_type=jnp.float32)
    v_ref[...] = jnp.dot(xt, w2t, preferred_element_type=jnp.float32)


def _tc_embed(xt, wt):
    return pl.pallas_call(
        _tc_embed_body,
        out_shape=[
            jax.ShapeDtypeStruct((_N, _COUT), jnp.float32),
            jax.ShapeDtypeStruct((_N, _COUT), jnp.float32),
        ],
    )(xt, wt)


def _sc_body(u_hbm, v_hbm, ii_hbm, jj_hbm, b_hbm, out_hbm,
             iv0, iv1, iv2, iv3, iv4, iv5, iv6, iv7,
             jv0, jv1, jv2, jv3, jv4, jv5, jv6, jv7,
             ur0, ur1, ur2, ur3, vr0, vr1, vr2, vr3,
             ovb, bv,
             sem_i0, sem_i1, sem_i2, sem_i3,
             sem_i4, sem_i5, sem_i6, sem_i7,
             sem_g0, sem_g1, sem_g2, sem_g3, sem_ob):
    cid = lax.axis_index("c")
    sid = lax.axis_index("s")
    n0 = jnp.where(cid == 0, sid * _NB0, _NS * _NB0 + sid * _NB1)
    nb = jnp.where(cid == 0, _NB0, _NB1)
    qt = jnp.where(cid == 0, _QT0, _QT1)
    pltpu.sync_copy(b_hbm, bv)
    bvecs = [bv[pl.ds(c * 16, 16)] for c in range(_LC)]
    neg = jnp.full((16,), -jnp.inf, jnp.float32)

    ivs = (iv0, iv1, iv2, iv3, iv4, iv5, iv6, iv7)
    jvs = (jv0, jv1, jv2, jv3, jv4, jv5, jv6, jv7)
    sem_i = (sem_i0, sem_i1, sem_i2, sem_i3,
             sem_i4, sem_i5, sem_i6, sem_i7)
    urs = (ur0, ur1, ur2, ur3)
    vrs = (vr0, vr1, vr2, vr3)
    sem_g = (sem_g0, sem_g1, sem_g2, sem_g3)

    def fetch_idx(k, chunk):
        base = (n0 + chunk * _CN) * _K
        pltpu.async_copy(ii_hbm.at[pl.ds(base, _ROWS)], ivs[k], sem_i[k])
        pltpu.async_copy(jj_hbm.at[pl.ds(base, _ROWS)], jvs[k], sem_i[k])

    def wait_idx(k):
        pltpu.make_async_copy(
            ii_hbm.at[pl.ds(0, _ROWS)], ivs[k], sem_i[k]).wait()
        pltpu.make_async_copy(
            jj_hbm.at[pl.ds(0, _ROWS)], jvs[k], sem_i[k]).wait()

    def issue_gather(r, k):
        pltpu.async_copy(u_hbm.at[ivs[k]], urs[r], sem_g[r])
        pltpu.async_copy(v_hbm.at[jvs[k]], vrs[r], sem_g[r])

    def wait_gather(r, k):
        pltpu.make_async_copy(u_hbm.at[ivs[k]], urs[r], sem_g[r]).wait()
        pltpu.make_async_copy(v_hbm.at[jvs[k]], vrs[r], sem_g[r]).wait()

    def compute(r, g):
        ur, vr = urs[r], vrs[r]
        for n in range(_CN):
            def kbody(kk, accs, n=n):
                row = n * _K + kk
                return tuple(
                    jnp.maximum(accs[c],
                                ur[row, pl.ds(c * 16, 16)]
                                + vr[row, pl.ds(c * 16, 16)])
                    for c in range(_LC))
            accs = lax.fori_loop(0, _K, kbody, (neg,) * _LC)
            orow = g * _CN + n
            for c in range(_LC):
                ovb[orow, pl.ds(c * 16, 16)] = jnp.maximum(
                    accs[c] + bvecs[c], 0.0)

    for k in range(8):
        fetch_idx(k, k)
    for k in range(4):
        wait_idx(k)
        issue_gather(k, k)

    def body(t, carry):
        for j in range(8):
            g = 8 * t + j
            r = j % 4
            wait_gather(r, j)

            @pl.when(t < qt - 1)
            def _():
                fetch_idx(j, g + 8)

            compute(r, g)

            def refill():
                k2 = (j + 4) % 8
                wait_idx(k2)
                issue_gather(r, k2)

            if j < 4:
                refill()
            else:
                pl.when(t < qt - 1)(refill)
        return carry

    lax.fori_loop(0, qt, body, 0)

    @pl.when(cid == 0)
    def _():
        pltpu.async_copy(ovb.at[pl.ds(0, _NB0)],
                         out_hbm.at[pl.ds(n0, _NB0)], sem_ob).wait()

    @pl.when(cid != 0)
    def _():
        pltpu.async_copy(ovb.at[pl.ds(0, _NB1)],
                         out_hbm.at[pl.ds(n0, _NB1)], sem_ob).wait()


def _sc_gather(u, v, ii, jj, b):
    mesh = plsc.VectorSubcoreMesh(core_axis_name="c", subcore_axis_name="s")
    fn = functools.partial(
        pl.kernel,
        out_type=jax.ShapeDtypeStruct((_NPAD, _COUT), jnp.float32),
        mesh=mesh,
        scratch_types=(
            [pltpu.VMEM((_ROWS,), jnp.int32)] * 8
            + [pltpu.VMEM((_ROWS,), jnp.int32)] * 8
            + [pltpu.VMEM((_ROWS, _COUT), jnp.float32)] * 4
            + [pltpu.VMEM((_ROWS, _COUT), jnp.float32)] * 4
            + [pltpu.VMEM((_NBMAX, _COUT), jnp.float32)]
            + [pltpu.VMEM((_COUT,), jnp.float32)]
            + [pltpu.SemaphoreType.DMA] * 13
        ),
    )(_sc_body)
    return fn(u, v, ii, jj, b)


def kernel(x, edge_index, W, b):
    xt = jnp.transpose(x.reshape(_C, _N))
    wt = jnp.transpose(W)
    u, v = _tc_embed(xt, wt)

    ei = edge_index.astype(jnp.int32)
    ii = ei[1].reshape(_N * _K)
    jj = ei[0].reshape(_N * _K)
    pad = _NPAD * _K - _N * _K
    ii = jnp.pad(ii, (0, pad))
    jj = jnp.pad(jj, (0, pad))

    out = _sc_gather(u, v, ii, jj, b)
    out = jnp.transpose(out[:_N])
    return out.reshape(_B, _COUT, _N, 1)
